# baseline (speedup 1.0000x reference)
import functools

import jax
import jax.numpy as jnp
from jax import lax
from jax.experimental import pallas as pl
from jax.experimental.pallas import tpu as pltpu
from jax.experimental.pallas import tpu_sc as plsc

N = 131072
HROWS = N // 2
NDAYS = 256
NC = 2
NS = 16
L = 16
NW = NC * NS
CHUNK = N // NW
RCHUNK = HROWS // NW
STEPS = RCHUNK // L
UNROLL = 4
ECLAMP = 21.0


def _sc_body(x0_h, x1_h, t0_h, t1_h, w_hbm, d_hbm, pi_out, pres_out,
             x0a_v, x0b_v, x1a_v, x1b_v, t0a_v, t0b_v, t1a_v, t1b_v,
             w_v, d_v, acc, pres, pilocal, sem):
    wid = lax.axis_index("s") * NC + lax.axis_index("c")
    rbase = wid * RCHUNK
    ebase = wid * CHUNK

    copies = [
        pltpu.async_copy(x0_h.at[pl.ds(rbase, RCHUNK)], x0a_v, sem),
        pltpu.async_copy(x0_h.at[pl.ds(rbase + HROWS, RCHUNK)], x0b_v, sem),
        pltpu.async_copy(x1_h.at[pl.ds(rbase, RCHUNK)], x1a_v, sem),
        pltpu.async_copy(x1_h.at[pl.ds(rbase + HROWS, RCHUNK)], x1b_v, sem),
        pltpu.async_copy(t0_h.at[pl.ds(rbase, RCHUNK)], t0a_v, sem),
        pltpu.async_copy(t0_h.at[pl.ds(rbase + HROWS, RCHUNK)], t0b_v, sem),
        pltpu.async_copy(t1_h.at[pl.ds(rbase, RCHUNK)], t1a_v, sem),
        pltpu.async_copy(t1_h.at[pl.ds(rbase + HROWS, RCHUNK)], t1b_v, sem),
        pltpu.async_copy(w_hbm.at[pl.ds(ebase, CHUNK)], w_v, sem),
        pltpu.async_copy(d_hbm.at[pl.ds(ebase, CHUNK)], d_v, sem),
    ]

    zf = jnp.zeros((L,), jnp.float32)

    def zero_acc(j, _):
        base = j * (4 * L)
        acc[pl.ds(base, L)] = zf
        acc[pl.ds(base + L, L)] = zf
        acc[pl.ds(base + 2 * L, L)] = zf
        acc[pl.ds(base + 3 * L, L)] = zf
        return 0

    lax.fori_loop(0, (L * NDAYS) // (4 * L), zero_acc, 0)

    def zero_pres(j, _):
        pres[pl.ds(j * L, L)] = zf
        return 0

    lax.fori_loop(0, NDAYS // L, zero_pres, 0)

    for c in copies:
        c.wait()

    iota = lax.iota(jnp.int32, L)
    ones = jnp.full((L,), 1.0, jnp.float32)
    lane_base = iota * NDAYS

    def one_step(k):
        e0a = jnp.exp(jnp.minimum(-12.0 * x0a_v[pl.ds(k, L)], ECLAMP)) + 1.0
        e0b = jnp.exp(jnp.minimum(-12.0 * x0b_v[pl.ds(k, L)], ECLAMP)) + 1.0
        e1a = jnp.exp(jnp.minimum(-12.0 * x1a_v[pl.ds(k, L)], ECLAMP)) + 1.0
        e1b = jnp.exp(jnp.minimum(-12.0 * x1b_v[pl.ds(k, L)], ECLAMP)) + 1.0
        na = t0a_v[pl.ds(k, L)] * e0b + t0b_v[pl.ds(k, L)] * e0a
        nb = t1a_v[pl.ds(k, L)] * e1b + t1b_v[pl.ds(k, L)] * e1a
        da = e0a * e0b
        db = e1a * e1b
        r = 1.0 / (da * db)
        eidx = (k + iota) * 2
        oidx = eidx + 1
        we = plsc.load_gather(w_v, [eidx])
        wo = plsc.load_gather(w_v, [oidx])
        de = plsc.load_gather(d_v, [eidx])
        do = plsc.load_gather(d_v, [oidx])
        plsc.addupdate_scatter(acc, [lane_base + de], we * (na * db * r))
        plsc.addupdate_scatter(acc, [lane_base + do], wo * (nb * da * r))
        plsc.store_scatter(pres, [de], ones)
        plsc.store_scatter(pres, [do], ones)

    def step(j, _):
        for u in range(UNROLL):
            one_step((j * UNROLL + u) * L)
        return 0

    lax.fori_loop(0, STEPS // UNROLL, step, 0)

    def fold(b, _):
        parts = [acc[pl.ds(lane * NDAYS + b * L, L)] for lane in range(L)]
        while len(parts) > 1:
            parts = [parts[i] + parts[i + 1] for i in range(0, len(parts), 2)]
        pilocal[pl.ds(b * L, L)] = parts[0]
        return 0

    lax.fori_loop(0, NDAYS // L, fold, 0)

    pltpu.sync_copy(pilocal, pi_out.at[wid])
    pltpu.sync_copy(pres, pres_out.at[wid])


_sc_call = functools.partial(
    pl.kernel,
    out_type=(
        jax.ShapeDtypeStruct((NW, NDAYS), jnp.float32),
        jax.ShapeDtypeStruct((NW, NDAYS), jnp.float32),
    ),
    mesh=plsc.VectorSubcoreMesh(core_axis_name="c", subcore_axis_name="s"),
    compiler_params=pltpu.CompilerParams(needs_layout_passes=False),
    scratch_types=[
        pltpu.VMEM((RCHUNK,), jnp.float32),
        pltpu.VMEM((RCHUNK,), jnp.float32),
        pltpu.VMEM((RCHUNK,), jnp.float32),
        pltpu.VMEM((RCHUNK,), jnp.float32),
        pltpu.VMEM((RCHUNK,), jnp.float32),
        pltpu.VMEM((RCHUNK,), jnp.float32),
        pltpu.VMEM((RCHUNK,), jnp.float32),
        pltpu.VMEM((RCHUNK,), jnp.float32),
        pltpu.VMEM((CHUNK,), jnp.float32),
        pltpu.VMEM((CHUNK,), jnp.int32),
        pltpu.VMEM((L * NDAYS,), jnp.float32),
        pltpu.VMEM((NDAYS,), jnp.float32),
        pltpu.VMEM((NDAYS,), jnp.float32),
        pltpu.SemaphoreType.DMA,
    ],
)(_sc_body)


def _fin_body(pi_ref, pres_ref, o_ref):
    pi = jnp.sum(pi_ref[...], axis=0, keepdims=True)
    pres = jnp.sum(pres_ref[...], axis=0, keepdims=True)
    sum_pi = jnp.sum(pi)
    sum_pi2 = jnp.sum(pi * pi)
    ndays = jnp.sum(jnp.where(pres > 0.0, 1.0, 0.0))
    o_ref[0, 0] = -sum_pi * jnp.maximum(sum_pi, 0.0) / sum_pi2 / ndays


_finisher = pl.pallas_call(
    _fin_body,
    out_shape=jax.ShapeDtypeStruct((1, 1), jnp.float32),
    out_specs=pl.BlockSpec(memory_space=pltpu.SMEM),
)


def kernel(inputs, targets, weights, date):
    pi_part, pres_part = _sc_call(
        inputs[:, 0], inputs[:, 1], targets[:, 0], targets[:, 1],
        weights, date)
    return _finisher(pi_part, pres_part).reshape(())


# speedup vs baseline: 1.0747x; 1.0747x over previous
import functools

import jax
import jax.numpy as jnp
from jax import lax
from jax.experimental import pallas as pl
from jax.experimental.pallas import tpu as pltpu
from jax.experimental.pallas import tpu_sc as plsc

N = 131072
HROWS = N // 2
NDAYS = 256
NC = 2
NS = 16
L = 16
NW = NC * NS
CHUNK = N // NW
RCHUNK = HROWS // NW
STEPS = RCHUNK // L
UNROLL = 2
ECLAMP = 21.0


def _sc_body(x0_h, x1_h, t0_h, t1_h, w_hbm, d_hbm, pi_out, pres_out,
             x0a_v, x0b_v, x1a_v, x1b_v, t0a_v, t0b_v, t1a_v, t1b_v,
             w_v, d_v, acc, pres, pilocal, sem):
    wid = lax.axis_index("s") * NC + lax.axis_index("c")
    rbase = wid * RCHUNK
    ebase = wid * CHUNK

    copies = [
        pltpu.async_copy(x0_h.at[pl.ds(rbase, RCHUNK)], x0a_v, sem),
        pltpu.async_copy(x0_h.at[pl.ds(rbase + HROWS, RCHUNK)], x0b_v, sem),
        pltpu.async_copy(x1_h.at[pl.ds(rbase, RCHUNK)], x1a_v, sem),
        pltpu.async_copy(x1_h.at[pl.ds(rbase + HROWS, RCHUNK)], x1b_v, sem),
        pltpu.async_copy(t0_h.at[pl.ds(rbase, RCHUNK)], t0a_v, sem),
        pltpu.async_copy(t0_h.at[pl.ds(rbase + HROWS, RCHUNK)], t0b_v, sem),
        pltpu.async_copy(t1_h.at[pl.ds(rbase, RCHUNK)], t1a_v, sem),
        pltpu.async_copy(t1_h.at[pl.ds(rbase + HROWS, RCHUNK)], t1b_v, sem),
        pltpu.async_copy(w_hbm.at[pl.ds(ebase, CHUNK)], w_v, sem),
        pltpu.async_copy(d_hbm.at[pl.ds(ebase, CHUNK)], d_v, sem),
    ]

    zf = jnp.zeros((L,), jnp.float32)

    def zero_acc(j, _):
        base = j * (4 * L)
        acc[pl.ds(base, L)] = zf
        acc[pl.ds(base + L, L)] = zf
        acc[pl.ds(base + 2 * L, L)] = zf
        acc[pl.ds(base + 3 * L, L)] = zf
        return 0

    lax.fori_loop(0, (L * NDAYS) // (4 * L), zero_acc, 0)

    def zero_pres(j, _):
        pres[pl.ds(j * L, L)] = zf
        return 0

    lax.fori_loop(0, NDAYS // L, zero_pres, 0)

    for c in copies:
        c.wait()

    iota = lax.iota(jnp.int32, L)
    ones = jnp.full((L,), 1.0, jnp.float32)
    lane_base = iota * NDAYS

    def one_step(k):
        e0a = jnp.exp(jnp.minimum(-12.0 * x0a_v[pl.ds(k, L)], ECLAMP)) + 1.0
        e0b = jnp.exp(jnp.minimum(-12.0 * x0b_v[pl.ds(k, L)], ECLAMP)) + 1.0
        e1a = jnp.exp(jnp.minimum(-12.0 * x1a_v[pl.ds(k, L)], ECLAMP)) + 1.0
        e1b = jnp.exp(jnp.minimum(-12.0 * x1b_v[pl.ds(k, L)], ECLAMP)) + 1.0
        na = t0a_v[pl.ds(k, L)] * e0b + t0b_v[pl.ds(k, L)] * e0a
        nb = t1a_v[pl.ds(k, L)] * e1b + t1b_v[pl.ds(k, L)] * e1a
        da = e0a * e0b
        db = e1a * e1b
        r = 1.0 / (da * db)
        eidx = (k + iota) * 2
        oidx = eidx + 1
        we = plsc.load_gather(w_v, [eidx])
        wo = plsc.load_gather(w_v, [oidx])
        de = plsc.load_gather(d_v, [eidx])
        do = plsc.load_gather(d_v, [oidx])
        plsc.addupdate_scatter(acc, [lane_base + de], we * (na * db * r))
        plsc.addupdate_scatter(acc, [lane_base + do], wo * (nb * da * r))
        plsc.store_scatter(pres, [de], ones)
        plsc.store_scatter(pres, [do], ones)

    @plsc.parallel_loop(0, STEPS * L, L, unroll=UNROLL)
    def _main(k):
        one_step(k)

    def fold(b, _):
        parts = [acc[pl.ds(lane * NDAYS + b * L, L)] for lane in range(L)]
        while len(parts) > 1:
            parts = [parts[i] + parts[i + 1] for i in range(0, len(parts), 2)]
        pilocal[pl.ds(b * L, L)] = parts[0]
        return 0

    lax.fori_loop(0, NDAYS // L, fold, 0)

    pltpu.sync_copy(pilocal, pi_out.at[wid])
    pltpu.sync_copy(pres, pres_out.at[wid])


_sc_call = functools.partial(
    pl.kernel,
    out_type=(
        jax.ShapeDtypeStruct((NW, NDAYS), jnp.float32),
        jax.ShapeDtypeStruct((NW, NDAYS), jnp.float32),
    ),
    mesh=plsc.VectorSubcoreMesh(core_axis_name="c", subcore_axis_name="s"),
    compiler_params=pltpu.CompilerParams(needs_layout_passes=False),
    scratch_types=[
        pltpu.VMEM((RCHUNK,), jnp.float32),
        pltpu.VMEM((RCHUNK,), jnp.float32),
        pltpu.VMEM((RCHUNK,), jnp.float32),
        pltpu.VMEM((RCHUNK,), jnp.float32),
        pltpu.VMEM((RCHUNK,), jnp.float32),
        pltpu.VMEM((RCHUNK,), jnp.float32),
        pltpu.VMEM((RCHUNK,), jnp.float32),
        pltpu.VMEM((RCHUNK,), jnp.float32),
        pltpu.VMEM((CHUNK,), jnp.float32),
        pltpu.VMEM((CHUNK,), jnp.int32),
        pltpu.VMEM((L * NDAYS,), jnp.float32),
        pltpu.VMEM((NDAYS,), jnp.float32),
        pltpu.VMEM((NDAYS,), jnp.float32),
        pltpu.SemaphoreType.DMA,
    ],
)(_sc_body)


def _fin_body(pi_ref, pres_ref, o_ref):
    pi = jnp.sum(pi_ref[...], axis=0, keepdims=True)
    pres = jnp.sum(pres_ref[...], axis=0, keepdims=True)
    sum_pi = jnp.sum(pi)
    sum_pi2 = jnp.sum(pi * pi)
    ndays = jnp.sum(jnp.where(pres > 0.0, 1.0, 0.0))
    o_ref[0, 0] = -sum_pi * jnp.maximum(sum_pi, 0.0) / sum_pi2 / ndays


_finisher = pl.pallas_call(
    _fin_body,
    out_shape=jax.ShapeDtypeStruct((1, 1), jnp.float32),
    out_specs=pl.BlockSpec(memory_space=pltpu.SMEM),
)


def kernel(inputs, targets, weights, date):
    pi_part, pres_part = _sc_call(
        inputs[:, 0], inputs[:, 1], targets[:, 0], targets[:, 1],
        weights, date)
    return _finisher(pi_part, pres_part).reshape(())


# speedup vs baseline: 1.0748x; 1.0001x over previous
import functools

import jax
import jax.numpy as jnp
from jax import lax
from jax.experimental import pallas as pl
from jax.experimental.pallas import tpu as pltpu
from jax.experimental.pallas import tpu_sc as plsc

N = 131072
HROWS = N // 2
NDAYS = 256
NC = 2
NS = 16
L = 16
NW = NC * NS
CHUNK = N // NW
RCHUNK = HROWS // NW
STEPS = RCHUNK // L
UNROLL = 4
ECLAMP = 21.0


def _sc_body(x0_h, x1_h, t0_h, t1_h, w_hbm, d_hbm, pi_out, pres_out,
             x0a_v, x0b_v, x1a_v, x1b_v, t0a_v, t0b_v, t1a_v, t1b_v,
             w_v, d_v, acc, pres, pilocal, sem):
    wid = lax.axis_index("s") * NC + lax.axis_index("c")
    rbase = wid * RCHUNK
    ebase = wid * CHUNK

    copies = [
        pltpu.async_copy(x0_h.at[pl.ds(rbase, RCHUNK)], x0a_v, sem),
        pltpu.async_copy(x0_h.at[pl.ds(rbase + HROWS, RCHUNK)], x0b_v, sem),
        pltpu.async_copy(x1_h.at[pl.ds(rbase, RCHUNK)], x1a_v, sem),
        pltpu.async_copy(x1_h.at[pl.ds(rbase + HROWS, RCHUNK)], x1b_v, sem),
        pltpu.async_copy(t0_h.at[pl.ds(rbase, RCHUNK)], t0a_v, sem),
        pltpu.async_copy(t0_h.at[pl.ds(rbase + HROWS, RCHUNK)], t0b_v, sem),
        pltpu.async_copy(t1_h.at[pl.ds(rbase, RCHUNK)], t1a_v, sem),
        pltpu.async_copy(t1_h.at[pl.ds(rbase + HROWS, RCHUNK)], t1b_v, sem),
        pltpu.async_copy(w_hbm.at[pl.ds(ebase, CHUNK)], w_v, sem),
        pltpu.async_copy(d_hbm.at[pl.ds(ebase, CHUNK)], d_v, sem),
    ]

    zf = jnp.zeros((L,), jnp.float32)

    def zero_acc(j, _):
        base = j * (4 * L)
        acc[pl.ds(base, L)] = zf
        acc[pl.ds(base + L, L)] = zf
        acc[pl.ds(base + 2 * L, L)] = zf
        acc[pl.ds(base + 3 * L, L)] = zf
        return 0

    lax.fori_loop(0, (L * NDAYS) // (4 * L), zero_acc, 0)

    def zero_pres(j, _):
        pres[pl.ds(j * L, L)] = zf
        return 0

    lax.fori_loop(0, NDAYS // L, zero_pres, 0)

    for c in copies:
        c.wait()

    iota = lax.iota(jnp.int32, L)
    ones = jnp.full((L,), 1.0, jnp.float32)
    lane_base = iota * NDAYS

    def one_step(k):
        e0a = jnp.exp(jnp.minimum(-12.0 * x0a_v[pl.ds(k, L)], ECLAMP)) + 1.0
        e0b = jnp.exp(jnp.minimum(-12.0 * x0b_v[pl.ds(k, L)], ECLAMP)) + 1.0
        e1a = jnp.exp(jnp.minimum(-12.0 * x1a_v[pl.ds(k, L)], ECLAMP)) + 1.0
        e1b = jnp.exp(jnp.minimum(-12.0 * x1b_v[pl.ds(k, L)], ECLAMP)) + 1.0
        na = t0a_v[pl.ds(k, L)] * e0b + t0b_v[pl.ds(k, L)] * e0a
        nb = t1a_v[pl.ds(k, L)] * e1b + t1b_v[pl.ds(k, L)] * e1a
        da = e0a * e0b
        db = e1a * e1b
        r = 1.0 / (da * db)
        eidx = (k + iota) * 2
        oidx = eidx + 1
        we = plsc.load_gather(w_v, [eidx])
        wo = plsc.load_gather(w_v, [oidx])
        de = plsc.load_gather(d_v, [eidx])
        do = plsc.load_gather(d_v, [oidx])
        plsc.addupdate_scatter(acc, [lane_base + de], we * (na * db * r))
        plsc.addupdate_scatter(acc, [lane_base + do], wo * (nb * da * r))
        plsc.store_scatter(pres, [de], ones)
        plsc.store_scatter(pres, [do], ones)

    @plsc.parallel_loop(0, STEPS * L, L, unroll=UNROLL)
    def _main(k):
        one_step(k)

    def fold(b, _):
        parts = [acc[pl.ds(lane * NDAYS + b * L, L)] for lane in range(L)]
        while len(parts) > 1:
            parts = [parts[i] + parts[i + 1] for i in range(0, len(parts), 2)]
        pilocal[pl.ds(b * L, L)] = parts[0]
        return 0

    lax.fori_loop(0, NDAYS // L, fold, 0)

    pltpu.sync_copy(pilocal, pi_out.at[wid])
    pltpu.sync_copy(pres, pres_out.at[wid])


_sc_call = functools.partial(
    pl.kernel,
    out_type=(
        jax.ShapeDtypeStruct((NW, NDAYS), jnp.float32),
        jax.ShapeDtypeStruct((NW, NDAYS), jnp.float32),
    ),
    mesh=plsc.VectorSubcoreMesh(core_axis_name="c", subcore_axis_name="s"),
    compiler_params=pltpu.CompilerParams(needs_layout_passes=False),
    scratch_types=[
        pltpu.VMEM((RCHUNK,), jnp.float32),
        pltpu.VMEM((RCHUNK,), jnp.float32),
        pltpu.VMEM((RCHUNK,), jnp.float32),
        pltpu.VMEM((RCHUNK,), jnp.float32),
        pltpu.VMEM((RCHUNK,), jnp.float32),
        pltpu.VMEM((RCHUNK,), jnp.float32),
        pltpu.VMEM((RCHUNK,), jnp.float32),
        pltpu.VMEM((RCHUNK,), jnp.float32),
        pltpu.VMEM((CHUNK,), jnp.float32),
        pltpu.VMEM((CHUNK,), jnp.int32),
        pltpu.VMEM((L * NDAYS,), jnp.float32),
        pltpu.VMEM((NDAYS,), jnp.float32),
        pltpu.VMEM((NDAYS,), jnp.float32),
        pltpu.SemaphoreType.DMA,
    ],
)(_sc_body)


def _fin_body(pi_ref, pres_ref, o_ref):
    pi = jnp.sum(pi_ref[...], axis=0, keepdims=True)
    pres = jnp.sum(pres_ref[...], axis=0, keepdims=True)
    sum_pi = jnp.sum(pi)
    sum_pi2 = jnp.sum(pi * pi)
    ndays = jnp.sum(jnp.where(pres > 0.0, 1.0, 0.0))
    o_ref[0, 0] = -sum_pi * jnp.maximum(sum_pi, 0.0) / sum_pi2 / ndays


_finisher = pl.pallas_call(
    _fin_body,
    out_shape=jax.ShapeDtypeStruct((1, 1), jnp.float32),
    out_specs=pl.BlockSpec(memory_space=pltpu.SMEM),
)


def kernel(inputs, targets, weights, date):
    pi_part, pres_part = _sc_call(
        inputs[:, 0], inputs[:, 1], targets[:, 0], targets[:, 1],
        weights, date)
    return _finisher(pi_part, pres_part).reshape(())


# speedup vs baseline: 1.0800x; 1.0048x over previous
import functools

import jax
import jax.numpy as jnp
from jax import lax
from jax.experimental import pallas as pl
from jax.experimental.pallas import tpu as pltpu
from jax.experimental.pallas import tpu_sc as plsc

N = 131072
HROWS = N // 2
NDAYS = 256
NC = 2
NS = 16
L = 16
NW = NC * NS
CHUNK = N // NW
RCHUNK = HROWS // NW
STEPS = RCHUNK // L
UNROLL = 4
ECLAMP = 21.0


def _sc_body(x0_h, x1_h, t0_h, t1_h, w_hbm, d_hbm, pi_out, pres_out,
             x0a_v, x0b_v, x1a_v, x1b_v, t0a_v, t0b_v, t1a_v, t1b_v,
             w_v, d_v, acc, pres, pilocal, sem):
    wid = lax.axis_index("s") * NC + lax.axis_index("c")
    rbase = wid * RCHUNK
    ebase = wid * CHUNK

    copies = [
        pltpu.async_copy(x0_h.at[pl.ds(rbase, RCHUNK)], x0a_v, sem),
        pltpu.async_copy(x0_h.at[pl.ds(rbase + HROWS, RCHUNK)], x0b_v, sem),
        pltpu.async_copy(x1_h.at[pl.ds(rbase, RCHUNK)], x1a_v, sem),
        pltpu.async_copy(x1_h.at[pl.ds(rbase + HROWS, RCHUNK)], x1b_v, sem),
        pltpu.async_copy(t0_h.at[pl.ds(rbase, RCHUNK)], t0a_v, sem),
        pltpu.async_copy(t0_h.at[pl.ds(rbase + HROWS, RCHUNK)], t0b_v, sem),
        pltpu.async_copy(t1_h.at[pl.ds(rbase, RCHUNK)], t1a_v, sem),
        pltpu.async_copy(t1_h.at[pl.ds(rbase + HROWS, RCHUNK)], t1b_v, sem),
        pltpu.async_copy(w_hbm.at[pl.ds(ebase, CHUNK)], w_v, sem),
        pltpu.async_copy(d_hbm.at[pl.ds(ebase, CHUNK)], d_v, sem),
    ]

    zf = jnp.zeros((L,), jnp.float32)

    @plsc.parallel_loop(0, L * NDAYS, L, unroll=8)
    def _zero_acc(j):
        acc[pl.ds(j, L)] = zf

    @plsc.parallel_loop(0, NDAYS, L, unroll=4)
    def _zero_pres(j):
        pres[pl.ds(j, L)] = zf

    for c in copies:
        c.wait()

    iota = lax.iota(jnp.int32, L)
    ones = jnp.full((L,), 1.0, jnp.float32)
    lane_base = iota * NDAYS

    def one_step(k):
        e0a = jnp.exp(jnp.minimum(-12.0 * x0a_v[pl.ds(k, L)], ECLAMP)) + 1.0
        e0b = jnp.exp(jnp.minimum(-12.0 * x0b_v[pl.ds(k, L)], ECLAMP)) + 1.0
        e1a = jnp.exp(jnp.minimum(-12.0 * x1a_v[pl.ds(k, L)], ECLAMP)) + 1.0
        e1b = jnp.exp(jnp.minimum(-12.0 * x1b_v[pl.ds(k, L)], ECLAMP)) + 1.0
        na = t0a_v[pl.ds(k, L)] * e0b + t0b_v[pl.ds(k, L)] * e0a
        nb = t1a_v[pl.ds(k, L)] * e1b + t1b_v[pl.ds(k, L)] * e1a
        da = e0a * e0b
        db = e1a * e1b
        r = 1.0 / (da * db)
        eidx = (k + iota) * 2
        oidx = eidx + 1
        we = plsc.load_gather(w_v, [eidx])
        wo = plsc.load_gather(w_v, [oidx])
        de = plsc.load_gather(d_v, [eidx])
        do = plsc.load_gather(d_v, [oidx])
        plsc.addupdate_scatter(acc, [lane_base + de], we * (na * db * r))
        plsc.addupdate_scatter(acc, [lane_base + do], wo * (nb * da * r))
        plsc.store_scatter(pres, [de], ones)
        plsc.store_scatter(pres, [do], ones)

    @plsc.parallel_loop(0, STEPS * L, L, unroll=UNROLL)
    def _main(k):
        one_step(k)

    @plsc.parallel_loop(0, NDAYS, L, unroll=2)
    def _fold(b):
        parts = [acc[pl.ds(lane * NDAYS + b, L)] for lane in range(L)]
        while len(parts) > 1:
            parts = [parts[i] + parts[i + 1] for i in range(0, len(parts), 2)]
        pilocal[pl.ds(b, L)] = parts[0]

    pltpu.sync_copy(pilocal, pi_out.at[wid])
    pltpu.sync_copy(pres, pres_out.at[wid])


_sc_call = functools.partial(
    pl.kernel,
    out_type=(
        jax.ShapeDtypeStruct((NW, NDAYS), jnp.float32),
        jax.ShapeDtypeStruct((NW, NDAYS), jnp.float32),
    ),
    mesh=plsc.VectorSubcoreMesh(core_axis_name="c", subcore_axis_name="s"),
    compiler_params=pltpu.CompilerParams(needs_layout_passes=False),
    scratch_types=[
        pltpu.VMEM((RCHUNK,), jnp.float32),
        pltpu.VMEM((RCHUNK,), jnp.float32),
        pltpu.VMEM((RCHUNK,), jnp.float32),
        pltpu.VMEM((RCHUNK,), jnp.float32),
        pltpu.VMEM((RCHUNK,), jnp.float32),
        pltpu.VMEM((RCHUNK,), jnp.float32),
        pltpu.VMEM((RCHUNK,), jnp.float32),
        pltpu.VMEM((RCHUNK,), jnp.float32),
        pltpu.VMEM((CHUNK,), jnp.float32),
        pltpu.VMEM((CHUNK,), jnp.int32),
        pltpu.VMEM((L * NDAYS,), jnp.float32),
        pltpu.VMEM((NDAYS,), jnp.float32),
        pltpu.VMEM((NDAYS,), jnp.float32),
        pltpu.SemaphoreType.DMA,
    ],
)(_sc_body)


def _fin_body(pi_ref, pres_ref, o_ref):
    pi = jnp.sum(pi_ref[...], axis=0, keepdims=True)
    pres = jnp.sum(pres_ref[...], axis=0, keepdims=True)
    sum_pi = jnp.sum(pi)
    sum_pi2 = jnp.sum(pi * pi)
    ndays = jnp.sum(jnp.where(pres > 0.0, 1.0, 0.0))
    o_ref[0, 0] = -sum_pi * jnp.maximum(sum_pi, 0.0) / sum_pi2 / ndays


_finisher = pl.pallas_call(
    _fin_body,
    out_shape=jax.ShapeDtypeStruct((1, 1), jnp.float32),
    out_specs=pl.BlockSpec(memory_space=pltpu.SMEM),
)


def kernel(inputs, targets, weights, date):
    pi_part, pres_part = _sc_call(
        inputs[:, 0], inputs[:, 1], targets[:, 0], targets[:, 1],
        weights, date)
    return _finisher(pi_part, pres_part).reshape(())


# speedup vs baseline: 1.0806x; 1.0006x over previous
import functools

import jax
import jax.numpy as jnp
from jax import lax
from jax.experimental import pallas as pl
from jax.experimental.pallas import tpu as pltpu
from jax.experimental.pallas import tpu_sc as plsc

N = 131072
HROWS = N // 2
NDAYS = 256
NC = 2
NS = 16
L = 16
NW = NC * NS
CHUNK = N // NW
RCHUNK = HROWS // NW
STEPS = RCHUNK // L
UNROLL = 4
ECLAMP = 21.0


def _sc_body(x0_h, x1_h, t0_h, t1_h, w_hbm, d_hbm, pi_out, pres_out,
             x0a_v, x0b_v, x1a_v, x1b_v, t0a_v, t0b_v, t1a_v, t1b_v,
             w_v, d_v, acc, pres, pilocal, sem):
    wid = lax.axis_index("s") * NC + lax.axis_index("c")
    rbase = wid * RCHUNK
    ebase = wid * CHUNK

    copies = [
        pltpu.async_copy(x0_h.at[pl.ds(rbase, RCHUNK)], x0a_v, sem),
        pltpu.async_copy(x0_h.at[pl.ds(rbase + HROWS, RCHUNK)], x0b_v, sem),
        pltpu.async_copy(x1_h.at[pl.ds(rbase, RCHUNK)], x1a_v, sem),
        pltpu.async_copy(x1_h.at[pl.ds(rbase + HROWS, RCHUNK)], x1b_v, sem),
        pltpu.async_copy(t0_h.at[pl.ds(rbase, RCHUNK)], t0a_v, sem),
        pltpu.async_copy(t0_h.at[pl.ds(rbase + HROWS, RCHUNK)], t0b_v, sem),
        pltpu.async_copy(t1_h.at[pl.ds(rbase, RCHUNK)], t1a_v, sem),
        pltpu.async_copy(t1_h.at[pl.ds(rbase + HROWS, RCHUNK)], t1b_v, sem),
        pltpu.async_copy(w_hbm.at[pl.ds(ebase, CHUNK)], w_v, sem),
        pltpu.async_copy(d_hbm.at[pl.ds(ebase, CHUNK)], d_v, sem),
    ]

    zf = jnp.zeros((L,), jnp.float32)

    @plsc.parallel_loop(0, L * NDAYS, L, unroll=8)
    def _zero_acc(j):
        acc[pl.ds(j, L)] = zf

    @plsc.parallel_loop(0, NDAYS, L, unroll=4)
    def _zero_pres(j):
        pres[pl.ds(j, L)] = zf

    for c in copies:
        c.wait()

    iota = lax.iota(jnp.int32, L)
    ones = jnp.full((L,), 1.0, jnp.float32)
    lane_base = iota * NDAYS

    def one_step(k):
        e0a = jnp.exp(jnp.minimum(-12.0 * x0a_v[pl.ds(k, L)], ECLAMP)) + 1.0
        e0b = jnp.exp(jnp.minimum(-12.0 * x0b_v[pl.ds(k, L)], ECLAMP)) + 1.0
        e1a = jnp.exp(jnp.minimum(-12.0 * x1a_v[pl.ds(k, L)], ECLAMP)) + 1.0
        e1b = jnp.exp(jnp.minimum(-12.0 * x1b_v[pl.ds(k, L)], ECLAMP)) + 1.0
        na = t0a_v[pl.ds(k, L)] * e0b + t0b_v[pl.ds(k, L)] * e0a
        nb = t1a_v[pl.ds(k, L)] * e1b + t1b_v[pl.ds(k, L)] * e1a
        da = e0a * e0b
        db = e1a * e1b
        r = 1.0 / (da * db)
        eidx = (k + iota) * 2
        oidx = eidx + 1
        we = plsc.load_gather(w_v, [eidx])
        wo = plsc.load_gather(w_v, [oidx])
        de = plsc.load_gather(d_v, [eidx])
        do = plsc.load_gather(d_v, [oidx])
        plsc.addupdate_scatter(acc, [lane_base + de], we * (na * db * r))
        plsc.addupdate_scatter(acc, [lane_base + do], wo * (nb * da * r))
        plsc.store_scatter(pres, [de], ones)
        plsc.store_scatter(pres, [do], ones)

    @plsc.parallel_loop(0, STEPS * L, L, unroll=UNROLL)
    def _main(k):
        one_step(k)

    @plsc.parallel_loop(0, NDAYS, L, unroll=2)
    def _fold(b):
        parts = [acc[pl.ds(lane * NDAYS + b, L)] for lane in range(L)]
        while len(parts) > 1:
            parts = [parts[i] + parts[i + 1] for i in range(0, len(parts), 2)]
        pilocal[pl.ds(b, L)] = parts[0]

    pltpu.sync_copy(pilocal, pi_out.at[wid])
    pltpu.sync_copy(pres, pres_out.at[wid])


_sc_call = functools.partial(
    pl.kernel,
    out_type=(
        jax.ShapeDtypeStruct((NW, NDAYS), jnp.float32),
        jax.ShapeDtypeStruct((NW, NDAYS), jnp.float32),
    ),
    mesh=plsc.VectorSubcoreMesh(core_axis_name="c", subcore_axis_name="s"),
    compiler_params=pltpu.CompilerParams(
        needs_layout_passes=False, disable_bounds_checks=True,
        disable_semaphore_checks=True),
    scratch_types=[
        pltpu.VMEM((RCHUNK,), jnp.float32),
        pltpu.VMEM((RCHUNK,), jnp.float32),
        pltpu.VMEM((RCHUNK,), jnp.float32),
        pltpu.VMEM((RCHUNK,), jnp.float32),
        pltpu.VMEM((RCHUNK,), jnp.float32),
        pltpu.VMEM((RCHUNK,), jnp.float32),
        pltpu.VMEM((RCHUNK,), jnp.float32),
        pltpu.VMEM((RCHUNK,), jnp.float32),
        pltpu.VMEM((CHUNK,), jnp.float32),
        pltpu.VMEM((CHUNK,), jnp.int32),
        pltpu.VMEM((L * NDAYS,), jnp.float32),
        pltpu.VMEM((NDAYS,), jnp.float32),
        pltpu.VMEM((NDAYS,), jnp.float32),
        pltpu.SemaphoreType.DMA,
    ],
)(_sc_body)


def _fin_body(pi_ref, pres_ref, o_ref):
    pi = jnp.sum(pi_ref[...], axis=0, keepdims=True)
    pres = jnp.sum(pres_ref[...], axis=0, keepdims=True)
    sum_pi = jnp.sum(pi)
    sum_pi2 = jnp.sum(pi * pi)
    ndays = jnp.sum(jnp.where(pres > 0.0, 1.0, 0.0))
    o_ref[0, 0] = -sum_pi * jnp.maximum(sum_pi, 0.0) / sum_pi2 / ndays


_finisher = pl.pallas_call(
    _fin_body,
    out_shape=jax.ShapeDtypeStruct((1, 1), jnp.float32),
    out_specs=pl.BlockSpec(memory_space=pltpu.SMEM),
)


def kernel(inputs, targets, weights, date):
    pi_part, pres_part = _sc_call(
        inputs[:, 0], inputs[:, 1], targets[:, 0], targets[:, 1],
        weights, date)
    return _finisher(pi_part, pres_part).reshape(())
